# baseline (speedup 1.0000x reference)
import functools

import jax
import jax.numpy as jnp
from jax import lax
from jax.experimental import pallas as pl
from jax.experimental.pallas import tpu as pltpu
from jax.experimental.pallas import tpu_sc as plsc

B = 4096
C = 64
H = 16
W = 16
ROW = C * H * W


def _make_fused(nbuf=2):
    info = plsc.get_sparse_core_info()
    nc, ns = info.num_cores, info.num_subcores
    nw = nc * ns
    assert B % (8 * nw) == 0
    b_per_w = B // nw
    outer_n = b_per_w // nbuf
    mesh = plsc.VectorSubcoreMesh(core_axis_name="c", subcore_axis_name="s")

    @functools.partial(
        pl.kernel,
        mesh=mesh,
        out_type=jax.ShapeDtypeStruct((B * ROW,), jnp.float32),
        scratch_types=(
            [
                pltpu.VMEM((b_per_w,), jnp.int32),
                pltpu.VMEM((b_per_w, C), jnp.float32),
            ]
            + [pltpu.VMEM((ROW,), jnp.float32) for _ in range(2 * nbuf)]
            + [pltpu.SemaphoreType.DMA for _ in range(2 * nbuf + 1)]
        ),
        compiler_params=pltpu.CompilerParams(
            needs_layout_passes=False, use_tc_tiling_on_sc=False
        ),
    )
    def fused_k(idx_hbm, table_hbm, x_hbm, out_hbm, idx_v, rows_v, *rest):
        xin = rest[0:nbuf]
        xout = rest[nbuf : 2 * nbuf]
        isem = rest[2 * nbuf : 3 * nbuf]
        osem = rest[3 * nbuf : 4 * nbuf]
        gsem = rest[4 * nbuf]

        wid = lax.axis_index("s") * nc + lax.axis_index("c")
        base = wid * b_per_w

        pltpu.sync_copy(idx_hbm.at[pl.ds(base, b_per_w)], idx_v)
        pltpu.async_copy(table_hbm.at[idx_v], rows_v, gsem).wait()

        def start_in(row, b):
            pltpu.make_async_copy(
                x_hbm.at[pl.ds(row * ROW, ROW)], xin[b], isem[b]
            ).start()

        for b in range(nbuf):
            start_in(base + b, b)

        def row_compute(xin_b, xout_b, r):
            rv = jnp.full((16,), r, jnp.int32)

            @plsc.parallel_loop(0, C, 1, unroll=4)
            def _cc(cc):
                sev = plsc.load_gather(rows_v, [rv, jnp.full((16,), cc, jnp.int32)])
                off = cc * (H * W)
                for h in range(H):
                    sl = pl.ds(off + h * W, W)
                    xout_b[sl] = xin_b[sl] - sev

        def outer(o, carry):
            for b in range(nbuf):
                r = o * nbuf + b
                row = base + r
                pltpu.make_async_copy(
                    x_hbm.at[pl.ds(row * ROW, ROW)], xin[b], isem[b]
                ).wait()

                @pl.when(o > 0)
                def _wait_out():
                    pltpu.make_async_copy(
                        xout[b], out_hbm.at[pl.ds(row * ROW, ROW)], osem[b]
                    ).wait()

                row_compute(xin[b], xout[b], r)

                pltpu.make_async_copy(
                    xout[b], out_hbm.at[pl.ds(row * ROW, ROW)], osem[b]
                ).start()

                @pl.when(o < outer_n - 1)
                def _next_in():
                    start_in(row + nbuf, b)

            return carry

        lax.fori_loop(0, outer_n, outer, 0)

        for b in range(nbuf):
            pltpu.make_async_copy(
                xout[b], out_hbm.at[pl.ds((base + b) * ROW, ROW)], osem[b]
            ).wait()

    return fused_k


def kernel(x, y, log_det_jac, z, factors):
    y32 = y.astype(jnp.int32)
    out1 = _make_fused()(y32, factors, x.reshape(B * ROW))
    return (out1.reshape(x.shape), log_det_jac, z)


# speedup vs baseline: 3.9763x; 3.9763x over previous
import functools

import jax
import jax.numpy as jnp
from jax import lax
from jax.experimental import pallas as pl
from jax.experimental.pallas import tpu as pltpu
from jax.experimental.pallas import tpu_sc as plsc

B = 4096
C = 64
H = 16
W = 16
ROW = C * H * W
NF2 = 50000


def _make_fused(rows, nbuf=2):
    info = plsc.get_sparse_core_info()
    nc, ns = info.num_cores, info.num_subcores
    nw = nc * ns
    assert rows % (8 * nw) == 0
    b_per_w = rows // nw
    outer_n = b_per_w // nbuf
    mesh = plsc.VectorSubcoreMesh(core_axis_name="c", subcore_axis_name="s")

    @functools.partial(
        pl.kernel,
        mesh=mesh,
        out_type=jax.ShapeDtypeStruct((rows, ROW), jnp.float32),
        scratch_types=(
            [
                pltpu.VMEM((b_per_w,), jnp.int32),
                pltpu.VMEM((b_per_w,), jnp.int32),
                pltpu.VMEM((b_per_w, 128), jnp.float32),
            ]
            + [pltpu.VMEM((ROW,), jnp.float32) for _ in range(2 * nbuf)]
            + [pltpu.SemaphoreType.DMA for _ in range(2 * nbuf + 1)]
        ),
        compiler_params=pltpu.CompilerParams(needs_layout_passes=False),
    )
    def fused_k(idx_hbm, table_hbm, x_hbm, out_hbm, idx_v, half_v, rows_v, *rest):
        xin = rest[0:nbuf]
        xout = rest[nbuf : 2 * nbuf]
        isem = rest[2 * nbuf : 3 * nbuf]
        osem = rest[3 * nbuf : 4 * nbuf]
        gsem = rest[4 * nbuf]

        wid = lax.axis_index("s") * nc + lax.axis_index("c")
        base = wid * b_per_w

        pltpu.sync_copy(idx_hbm.at[pl.ds(base, b_per_w)], idx_v)
        for i in range(b_per_w // 16):
            half_v[pl.ds(i * 16, 16)] = lax.shift_right_logical(
                idx_v[pl.ds(i * 16, 16)], 1
            )
        pltpu.async_copy(table_hbm.at[half_v], rows_v, gsem).wait()

        for b in range(nbuf):
            pltpu.make_async_copy(x_hbm.at[base + b], xin[b], isem[b]).start()

        def row_compute(xin_b, xout_b, r):
            rv = jnp.full((16,), r, jnp.int32)
            yv = plsc.load_gather(idx_v, [rv])
            colbase = (yv & 1) * C

            @plsc.parallel_loop(0, C, 1, unroll=4)
            def _cc(cc):
                sev = plsc.load_gather(rows_v, [rv, colbase + cc])
                off = cc * (H * W)
                for h in range(H):
                    sl = pl.ds(off + h * W, W)
                    xout_b[sl] = xin_b[sl] - sev

        def outer(o, carry):
            for b in range(nbuf):
                r = o * nbuf + b
                row = base + r
                pltpu.make_async_copy(x_hbm.at[row], xin[b], isem[b]).wait()

                @pl.when(o > 0)
                def _wait_out():
                    pltpu.make_async_copy(
                        xout[b], out_hbm.at[row], osem[b]
                    ).wait()

                row_compute(xin[b], xout[b], r)

                pltpu.make_async_copy(xout[b], out_hbm.at[row], osem[b]).start()

                @pl.when(o < outer_n - 1)
                def _next_in():
                    pltpu.make_async_copy(
                        x_hbm.at[row + nbuf], xin[b], isem[b]
                    ).start()

            return carry

        lax.fori_loop(0, outer_n, outer, 0)

        for b in range(nbuf):
            pltpu.make_async_copy(
                xout[b], out_hbm.at[base + b], osem[b]
            ).wait()

    return fused_k


def kernel(x, y, log_det_jac, z, factors):
    y32 = y.astype(jnp.int32)
    table2 = factors.reshape(NF2, 128)
    x2 = x.reshape(B, ROW)
    half = B // 2
    f = _make_fused(half)
    outs = [
        f(y32[s * half : (s + 1) * half], table2, x2[s * half : (s + 1) * half])
        for s in range(2)
    ]
    out2 = jnp.concatenate(outs, axis=0)
    return (out2.reshape(x.shape), log_det_jac, z)


# speedup vs baseline: 5.5718x; 1.4013x over previous
import functools

import jax
import jax.numpy as jnp
from jax import lax
from jax.experimental import pallas as pl
from jax.experimental.pallas import tpu as pltpu
from jax.experimental.pallas import tpu_sc as plsc

B = 4096
C = 64
H = 16
W = 16
ROW = C * H * W
NF2 = 50000


def _make_fused(rows, nbuf=2):
    info = plsc.get_sparse_core_info()
    nc, ns = info.num_cores, info.num_subcores
    nw = nc * ns
    assert rows % (8 * nw) == 0
    b_per_w = rows // nw
    outer_n = b_per_w // nbuf
    mesh = plsc.VectorSubcoreMesh(core_axis_name="c", subcore_axis_name="s")

    @functools.partial(
        pl.kernel,
        mesh=mesh,
        out_type=jax.ShapeDtypeStruct((rows, ROW), jnp.float32),
        scratch_types=(
            [
                pltpu.VMEM((b_per_w,), jnp.int32),
                pltpu.VMEM((b_per_w,), jnp.int32),
                pltpu.VMEM((b_per_w, 128), jnp.float32),
            ]
            + [pltpu.VMEM((ROW,), jnp.float32) for _ in range(2 * nbuf)]
            + [pltpu.SemaphoreType.DMA for _ in range(2 * nbuf + 1)]
        ),
        compiler_params=pltpu.CompilerParams(needs_layout_passes=False),
    )
    def fused_k(idx_hbm, table_hbm, x_hbm, out_hbm, idx_v, half_v, rows_v, *rest):
        xin = rest[0:nbuf]
        xout = rest[nbuf : 2 * nbuf]
        isem = rest[2 * nbuf : 3 * nbuf]
        osem = rest[3 * nbuf : 4 * nbuf]
        gsem = rest[4 * nbuf]

        wid = lax.axis_index("s") * nc + lax.axis_index("c")
        base = wid * b_per_w

        pltpu.sync_copy(idx_hbm.at[pl.ds(base, b_per_w)], idx_v)
        for i in range(b_per_w // 16):
            half_v[pl.ds(i * 16, 16)] = lax.shift_right_logical(
                idx_v[pl.ds(i * 16, 16)], 1
            )
        pltpu.async_copy(table_hbm.at[half_v], rows_v, gsem).wait()

        for b in range(nbuf):
            pltpu.make_async_copy(x_hbm.at[base + b], xin[b], isem[b]).start()

        def row_compute(xin_b, xout_b, r):
            rv = jnp.full((16,), r, jnp.int32)
            yv = plsc.load_gather(idx_v, [rv])
            colbase = (yv & 1) * C

            @plsc.parallel_loop(0, C, 1, unroll=4)
            def _cc(cc):
                sev = plsc.load_gather(rows_v, [rv, colbase + cc])
                off = cc * (H * W)
                for h in range(H):
                    sl = pl.ds(off + h * W, W)
                    xout_b[sl] = xin_b[sl] - sev

        def outer(o, carry):
            for b in range(nbuf):
                r = o * nbuf + b
                row = base + r
                pltpu.make_async_copy(x_hbm.at[row], xin[b], isem[b]).wait()

                @pl.when(o > 0)
                def _wait_out():
                    pltpu.make_async_copy(
                        xout[b], out_hbm.at[row], osem[b]
                    ).wait()

                row_compute(xin[b], xout[b], r)

                pltpu.make_async_copy(xout[b], out_hbm.at[row], osem[b]).start()

                @pl.when(o < outer_n - 1)
                def _next_in():
                    pltpu.make_async_copy(
                        x_hbm.at[row + nbuf], xin[b], isem[b]
                    ).start()

            return carry

        lax.fori_loop(0, outer_n, outer, 0)

        for b in range(nbuf):
            pltpu.make_async_copy(
                xout[b], out_hbm.at[base + b], osem[b]
            ).wait()

    return fused_k


def kernel(x, y, log_det_jac, z, factors):
    y32 = y.astype(jnp.int32)
    table2 = factors.reshape(NF2, 128)
    x2 = x.reshape(B, ROW)
    out2 = _make_fused(B)(y32, table2, x2)
    return (out2.reshape(x.shape), log_det_jac, z)
